# Initial kernel scaffold; baseline (speedup 1.0000x reference)
import jax
import jax.numpy as jnp
from jax.experimental import pallas as pl


def kernel(speaker, x, edge_index, edge_norm, edge_type, seq_lengths, umask, w1, w2, Wl, bl, Ws, bs):
    raise NotImplementedError("write your pallas kernel here")


# speedup vs baseline: 17.4441x; 17.4441x over previous
import jax
import jax.numpy as jnp
from jax import lax
from jax.experimental import pallas as pl
from jax.experimental.pallas import tpu as pltpu
from jax.experimental.pallas import tpu_sc as plsc

_N = 10000
_E = 320000
_H = 128
_NC = 2
_NS = 16
_NW = _NC * _NS
_EPW = _E // _NW
_K = 80
_NCH = _EPW // _K
_NP = 10240
_RPS = _NP // _NS
_ZR = 160

_BN = 400
_NB = _N // _BN

_mesh = plsc.VectorSubcoreMesh(core_axis_name="c", subcore_axis_name="s")


def _fill(ref, nrows, value):
    def _b(t, carry):
        i = t // (_H // 16)
        j = (t % (_H // 16)) * 16
        ref[i, pl.ds(j, 16)] = jnp.full((16,), value, jnp.float32)
        return carry
    lax.fori_loop(0, nrows * (_H // 16), _b, 0)


def _zero_acc(acc, zero_v, sid):
    _fill(zero_v, _ZR, 0.0)

    def _zacc(c, carry):
        pltpu.sync_copy(zero_v, acc.at[pl.ds(sid * _RPS + c * _ZR, _ZR)])
        return carry
    lax.fori_loop(0, _RPS // _ZR, _zacc, 0)
    plsc.subcore_barrier()


def _dump_acc(acc, out_hbm, cid, sid):
    plsc.subcore_barrier()
    pltpu.sync_copy(acc.at[pl.ds(sid * _RPS, _RPS)],
                    out_hbm.at[pl.ds(cid * _NP + sid * _RPS, _RPS)])


def _seg_body(tbl_hbm, dst_hbm, en_hbm, row_hbm, out_hbm,
              idx_v, dst_v, en_v, rows_v, zero_v, acc, sem):
    cid = lax.axis_index("c")
    sid = lax.axis_index("s")
    wid = sid * _NC + cid
    _zero_acc(acc, zero_v, sid)
    base = wid * _EPW

    def _chunk(ci, carry):
        off = base + ci * _K
        pltpu.sync_copy(row_hbm.at[pl.ds(off, _K)], idx_v)
        pltpu.sync_copy(dst_hbm.at[pl.ds(off, _K)], dst_v)
        pltpu.sync_copy(en_hbm.at[pl.ds(off, _K)], en_v)
        pltpu.async_copy(tbl_hbm.at[idx_v], rows_v, sem).wait()

        def _grp(g, c1):
            en16 = en_v[pl.ds(g * 16, 16)]

            def _scale(kk, c2):
                k = g * 16 + kk
                s = lax.gather(
                    en16, jnp.broadcast_to(kk, (16, 1)),
                    dimension_numbers=lax.GatherDimensionNumbers(
                        offset_dims=(), collapsed_slice_dims=(0,),
                        start_index_map=(0,)),
                    slice_sizes=(1,),
                    mode=lax.GatherScatterMode.PROMISE_IN_BOUNDS)

                def _mul(j, c3):
                    rows_v[k, pl.ds(j * 16, 16)] = (
                        rows_v[k, pl.ds(j * 16, 16)] * s)
                    return c3
                lax.fori_loop(0, _H // 16, _mul, 0)
                return c2
            lax.fori_loop(0, 16, _scale, 0)
            return c1
        lax.fori_loop(0, _K // 16, _grp, 0)

        pltpu.sync_copy(rows_v, acc.at[dst_v], add=True)
        return carry
    lax.fori_loop(0, _NCH, _chunk, 0)
    _dump_acc(acc, out_hbm, cid, sid)


_seg_kernel = pl.kernel(
    _seg_body,
    mesh=_mesh,
    out_type=jax.ShapeDtypeStruct((_NC * _NP, _H), jnp.float32),
    scratch_types=[
        pltpu.VMEM((_K,), jnp.int32),
        pltpu.VMEM((_K,), jnp.int32),
        pltpu.VMEM((_K,), jnp.float32),
        pltpu.VMEM((_K, _H), jnp.float32),
        pltpu.VMEM((_ZR, _H), jnp.float32),
        pltpu.VMEM_SHARED((_NP, _H), jnp.float32),
        pltpu.SemaphoreType.DMA,
    ],
)


def _cnt_body(dst_hbm, out_hbm, dst_v, ones_v, zero_v, acc):
    cid = lax.axis_index("c")
    sid = lax.axis_index("s")
    wid = sid * _NC + cid
    _zero_acc(acc, zero_v, sid)
    _fill(ones_v, _K, 1.0)
    base = wid * _EPW

    def _chunk(ci, carry):
        pltpu.sync_copy(dst_hbm.at[pl.ds(base + ci * _K, _K)], dst_v)
        pltpu.sync_copy(ones_v, acc.at[dst_v], add=True)
        return carry
    lax.fori_loop(0, _NCH, _chunk, 0)
    _dump_acc(acc, out_hbm, cid, sid)


_cnt_kernel = pl.kernel(
    _cnt_body,
    mesh=_mesh,
    out_type=jax.ShapeDtypeStruct((_NC * _NP, _H), jnp.float32),
    scratch_types=[
        pltpu.VMEM((_K,), jnp.int32),
        pltpu.VMEM((_K, _H), jnp.float32),
        pltpu.VMEM((_ZR, _H), jnp.float32),
        pltpu.VMEM_SHARED((_NP, _H), jnp.float32),
    ],
)


def _mm_body(x_ref, w_ref, o_ref):
    o_ref[0] = jnp.dot(x_ref[...], w_ref[0],
                       preferred_element_type=jnp.float32)


def _table1(x, w):
    return pl.pallas_call(
        _mm_body,
        grid=(2, _NB),
        in_specs=[
            pl.BlockSpec((_BN, _H), lambda r, b: (b, 0)),
            pl.BlockSpec((1, _H, _H), lambda r, b: (r, 0, 0)),
        ],
        out_specs=pl.BlockSpec((1, _BN, _H), lambda r, b: (r, b, 0)),
        out_shape=jax.ShapeDtypeStruct((2, _N, _H), jnp.float32),
    )(x, w)


def _combine_mm_body(acc_ref, cnt_ref, w_ref, o_ref):
    a = acc_ref[0] + acc_ref[1]
    cnt = jnp.maximum(cnt_ref[0, :, :1] + cnt_ref[1, :, :1], 1.0)
    o1 = jax.nn.sigmoid(a / cnt)
    o_ref[0] = jnp.dot(o1, w_ref[0], preferred_element_type=jnp.float32)


def _table2(acc, cnt, w):
    return pl.pallas_call(
        _combine_mm_body,
        grid=(2, _NB),
        in_specs=[
            pl.BlockSpec((2, _BN, _H), lambda r, b: (0, b, 0)),
            pl.BlockSpec((2, _BN, _H), lambda r, b: (0, b, 0)),
            pl.BlockSpec((1, _H, _H), lambda r, b: (r, 0, 0)),
        ],
        out_specs=pl.BlockSpec((1, _BN, _H), lambda r, b: (r, b, 0)),
        out_shape=jax.ShapeDtypeStruct((2, _N, _H), jnp.float32),
    )(acc, cnt, w)


def _head_body(x_ref, acc_ref, cnt_ref, wl_ref, bl_ref, ws_ref, bs_ref,
               lp_ref, em_ref):
    a = acc_ref[0] + acc_ref[1]
    cnt = jnp.maximum(cnt_ref[0, :, :1] + cnt_ref[1, :, :1], 1.0)
    out2 = a / cnt
    em = jnp.concatenate([x_ref[...], out2], axis=1)
    em_ref[...] = em
    hidden = jax.nn.relu(
        jnp.dot(em, wl_ref[...], preferred_element_type=jnp.float32)
        + bl_ref[...])
    logits = jnp.dot(hidden, ws_ref[...],
                     preferred_element_type=jnp.float32) + bs_ref[...]
    m = jnp.max(logits, axis=1, keepdims=True)
    lse = m + jnp.log(jnp.sum(jnp.exp(logits - m), axis=1, keepdims=True))
    lp_ref[...] = logits - lse


def _head(x, acc, cnt, Wl, bl, Ws, bs):
    C = Ws.shape[1]
    return pl.pallas_call(
        _head_body,
        grid=(_NB,),
        in_specs=[
            pl.BlockSpec((_BN, _H), lambda b: (b, 0)),
            pl.BlockSpec((2, _BN, _H), lambda b: (0, b, 0)),
            pl.BlockSpec((2, _BN, _H), lambda b: (0, b, 0)),
            pl.BlockSpec((2 * _H, _H), lambda b: (0, 0)),
            pl.BlockSpec((1, _H), lambda b: (0, 0)),
            pl.BlockSpec((_H, C), lambda b: (0, 0)),
            pl.BlockSpec((1, C), lambda b: (0, 0)),
        ],
        out_specs=[
            pl.BlockSpec((_BN, C), lambda b: (b, 0)),
            pl.BlockSpec((_BN, 2 * _H), lambda b: (b, 0)),
        ],
        out_shape=[
            jax.ShapeDtypeStruct((_N, C), jnp.float32),
            jax.ShapeDtypeStruct((_N, 2 * _H), jnp.float32),
        ],
    )(x, acc, cnt, Wl, bl.reshape(1, _H), Ws, bs.reshape(1, C))


def kernel(speaker, x, edge_index, edge_norm, edge_type, seq_lengths, umask,
           w1, w2, Wl, bl, Ws, bs):
    rowidx = edge_type * _N + edge_index[1]
    dstidx = edge_index[0]
    en2 = edge_norm * 2.0

    cnt = _cnt_kernel(dstidx).reshape(_NC, _NP, _H)[:, :_N]
    tbl1 = _table1(x, w1).reshape(2 * _N, _H)
    acc1 = _seg_kernel(tbl1, dstidx, en2, rowidx).reshape(_NC, _NP, _H)[:, :_N]
    tbl2 = _table2(acc1, cnt, w2).reshape(2 * _N, _H)
    acc2 = _seg_kernel(tbl2, dstidx, en2, rowidx).reshape(_NC, _NP, _H)[:, :_N]
    log_prob, emotions = _head(x, acc2, cnt, Wl, bl, Ws, bs)
    return (log_prob, x, emotions)
